# Initial kernel scaffold; baseline (speedup 1.0000x reference)
import jax
import jax.numpy as jnp
from jax.experimental import pallas as pl


def kernel(z, in_proj_v, in_proj_g, in_proj_b, codebook, out_proj_v, out_proj_g, out_proj_b):
    raise NotImplementedError("write your pallas kernel here")


# speedup vs baseline: 1.2616x; 1.2616x over previous
import functools

import jax
import jax.numpy as jnp
from jax import lax
from jax.experimental import pallas as pl
from jax.experimental.pallas import tpu as pltpu
from jax.experimental.pallas import tpu_sc as plsc

B, C_IN, T = 8, 1024, 2048
K, D = 8192, 32
BT = 512
BK = 1024
NTB = T // BT
NKB = K // BK


def _head_body(zin_ref, cb_ref, idx_ref,
               cbnh_scr, cbsq_scr, ench_scr, encsq_scr,
               best_scr, bidx_scr):
    b = pl.program_id(0)
    t = pl.program_id(1)
    k = pl.program_id(2)

    @pl.when((b == 0) & (t == 0) & (k == 0))
    def _init_consts():
        cb = cb_ref[...]
        cn = jnp.sqrt(jnp.sum(cb * cb, axis=1, keepdims=True))
        cbn = cb / jnp.maximum(cn, 1e-12)
        cbnh_scr[...] = cbn.astype(jnp.bfloat16)
        cbsq_scr[...] = jnp.sum(cbn * cbn, axis=1, keepdims=True)

    @pl.when(k == 0)
    def _normalize():
        zi = zin_ref[0]
        s = jnp.sum(zi * zi, axis=0, keepdims=True)
        encn = zi / jnp.maximum(jnp.sqrt(s), 1e-12)
        ench_scr[...] = encn.astype(jnp.bfloat16)
        encsq_scr[...] = jnp.sum(encn * encn, axis=0, keepdims=True)
        best_scr[...] = jnp.full((1, BT), -jnp.inf, jnp.float32)
        bidx_scr[...] = jnp.zeros((1, BT), jnp.int32)

    cbh_k = cbnh_scr[pl.ds(k * BK, BK), :]
    cbsq_k = cbsq_scr[pl.ds(k * BK, BK), :]
    prod = jnp.dot(cbh_k, ench_scr[...],
                   preferred_element_type=jnp.float32)
    neg = (2.0 * prod - encsq_scr[...]) - cbsq_k
    bmax = jnp.max(neg, axis=0, keepdims=True)
    rows = lax.broadcasted_iota(jnp.int32, (BK, BT), 0)
    lidx = jnp.min(jnp.where(neg == bmax, rows, K),
                   axis=0, keepdims=True)
    gidx = lidx + k * BK
    better = bmax > best_scr[...]
    best_scr[...] = jnp.where(better, bmax, best_scr[...])
    bidx_scr[...] = jnp.where(better, gidx, bidx_scr[...])

    @pl.when(k == NKB - 1)
    def _emit_idx():
        idx_ref[0, 0, 0] = bidx_scr[0]


def _head_call(zi, cb):
    return pl.pallas_call(
        _head_body,
        grid=(B, NTB, NKB),
        in_specs=[
            pl.BlockSpec((1, D, BT), lambda b, t, k: (b, 0, t)),
            pl.BlockSpec((K, D), lambda b, t, k: (0, 0)),
        ],
        out_specs=pl.BlockSpec((1, 1, 1, BT), lambda b, t, k: (b, t, 0, 0)),
        out_shape=jax.ShapeDtypeStruct((B, NTB, 1, BT), jnp.int32),
        scratch_shapes=[
            pltpu.VMEM((K, D), jnp.bfloat16),
            pltpu.VMEM((K, 1), jnp.float32),
            pltpu.VMEM((D, BT), jnp.bfloat16),
            pltpu.VMEM((1, BT), jnp.float32),
            pltpu.VMEM((1, BT), jnp.float32),
            pltpu.VMEM((1, BT), jnp.int32),
        ],
        compiler_params=pltpu.CompilerParams(
            dimension_semantics=("arbitrary", "arbitrary", "arbitrary"),
        ),
    )(zi, cb)


def _sc_gather(table, idx_flat):
    info = plsc.get_sparse_core_info()
    nw = info.num_cores * info.num_subcores
    b_per_w = (B * T) // nw
    mesh = plsc.VectorSubcoreMesh(core_axis_name="c", subcore_axis_name="s")

    @functools.partial(
        pl.kernel,
        out_type=jax.ShapeDtypeStruct((B * T, D), jnp.float32),
        mesh=mesh,
        compiler_params=pltpu.CompilerParams(use_tc_tiling_on_sc=False),
        scratch_types=[
            pltpu.VMEM((b_per_w,), jnp.int32),
            pltpu.VMEM((b_per_w, D), jnp.float32),
            pltpu.SemaphoreType.DMA,
        ],
    )
    def gather_kernel(table_hbm, idx_hbm, out_hbm, idx_v, rows_v, sem):
        wid = lax.axis_index("s") * info.num_cores + lax.axis_index("c")
        base = wid * b_per_w
        pltpu.sync_copy(idx_hbm.at[pl.ds(base, b_per_w)], idx_v)
        pltpu.async_copy(table_hbm.at[idx_v], rows_v, sem).wait()
        pltpu.sync_copy(rows_v, out_hbm.at[pl.ds(base, b_per_w)])

    return gather_kernel(table, idx_flat)


def _tail_body(zqr_ref, zi_ref, eye_ref, vout_ref, gout_ref, bout_ref,
               zq_ref, zo_ref, wout_scr):
    b = pl.program_id(0)
    t = pl.program_id(1)

    @pl.when((b == 0) & (t == 0))
    def _init_wout():
        v = vout_ref[...]
        n = jnp.sqrt(jnp.sum(v * v, axis=1, keepdims=True))
        wout_scr[...] = (gout_ref[...] * v) / n

    zq_rows = zqr_ref[...]
    zqt = lax.dot_general(eye_ref[...], zq_rows,
                          (((0,), (1,)), ((), ())),
                          precision=lax.Precision.HIGHEST,
                          preferred_element_type=jnp.float32)
    zi = zi_ref[0]
    zq_st = zi + (zqt - zi)
    zq_ref[0] = zq_st
    zo = jnp.dot(wout_scr[...].astype(jnp.bfloat16),
                 zq_st.astype(jnp.bfloat16),
                 preferred_element_type=jnp.float32) + bout_ref[...]
    zo_ref[0] = zo


def _tail_call(zq_rows, zi, eye, vout, gout2, bout2):
    return pl.pallas_call(
        _tail_body,
        grid=(B, NTB),
        in_specs=[
            pl.BlockSpec((BT, D), lambda b, t: (b * NTB + t, 0)),
            pl.BlockSpec((1, D, BT), lambda b, t: (b, 0, t)),
            pl.BlockSpec((D, D), lambda b, t: (0, 0)),
            pl.BlockSpec((C_IN, D), lambda b, t: (0, 0)),
            pl.BlockSpec((C_IN, 1), lambda b, t: (0, 0)),
            pl.BlockSpec((C_IN, 1), lambda b, t: (0, 0)),
        ],
        out_specs=[
            pl.BlockSpec((1, D, BT), lambda b, t: (b, 0, t)),
            pl.BlockSpec((1, C_IN, BT), lambda b, t: (b, 0, t)),
        ],
        out_shape=[
            jax.ShapeDtypeStruct((B, D, T), jnp.float32),
            jax.ShapeDtypeStruct((B, C_IN, T), jnp.float32),
        ],
        scratch_shapes=[
            pltpu.VMEM((C_IN, D), jnp.float32),
        ],
        compiler_params=pltpu.CompilerParams(
            dimension_semantics=("arbitrary", "arbitrary"),
        ),
    )(zq_rows, zi, eye, vout, gout2, bout2)


def kernel(z, in_proj_v, in_proj_g, in_proj_b, codebook,
           out_proj_v, out_proj_g, out_proj_b):
    gout2 = out_proj_g.reshape(C_IN, 1)
    bout2 = out_proj_b.reshape(C_IN, 1)
    eye = jnp.eye(D, dtype=jnp.float32)

    norm = jnp.linalg.norm(in_proj_v, axis=1, keepdims=True)
    w_in = in_proj_g[:, None] * in_proj_v / norm
    z_i = jnp.einsum("dc,bct->bdt", w_in, z) + in_proj_b[None, :, None]

    idx4 = _head_call(z_i, codebook)
    idx_flat = idx4.reshape(B * T)
    zq_rows = _sc_gather(codebook, idx_flat)
    z_q, z_o = _tail_call(zq_rows, z_i, eye, out_proj_v, gout2, bout2)
    indices = idx4.reshape(B, T)
    return (z_i, z_q, z_o, indices)
